# baseline (speedup 1.0000x reference)
import functools
import jax
import jax.numpy as jnp
from jax import lax
from jax.experimental import pallas as pl
from jax.experimental.pallas import tpu as pltpu
from jax.experimental.pallas import tpu_sc as plsc

VOCAB = 100000
NADD = 1024
D = 64
BATCH = 4096
HIST = 200
B = BATCH * HIST
NC, NS = 2, 16
NW = NC * NS
BPW = B // NW
CH = 256
NCHUNK = BPW // CH
GW = 128
NSUB = CH // GW

_mesh = plsc.VectorSubcoreMesh(core_axis_name="c", subcore_axis_name="s")


@functools.partial(
    pl.kernel,
    mesh=_mesh,
    out_type=jax.ShapeDtypeStruct((B, D), jnp.float32),
    scratch_types=[
        pltpu.VMEM((CH,), jnp.int32),
        pltpu.VMEM((CH, D), jnp.float32),
        pltpu.SemaphoreType.DMA,
        pltpu.SemaphoreType.DMA,
    ],
    compiler_params=pltpu.CompilerParams(use_tc_tiling_on_sc=False),
)
def _gather_kernel(table_hbm, idx_hbm, out_hbm, idx_v, rows_v, gsem, osem):
    wid = lax.axis_index("s") * NC + lax.axis_index("c")
    base = wid * BPW

    def chunk_body(c, _):
        cbase = base + c * CH
        pltpu.sync_copy(idx_hbm.at[pl.ds(cbase, CH)], idx_v)
        for j in range(NSUB):
            pltpu.async_copy(
                table_hbm.at[idx_v.at[pl.ds(j * GW, GW)]],
                rows_v.at[pl.ds(j * GW, GW)],
                gsem,
            )
        for j in range(NSUB):
            pltpu.make_async_copy(
                table_hbm.at[idx_v.at[pl.ds(j * GW, GW)]],
                rows_v.at[pl.ds(j * GW, GW)],
                gsem,
            ).wait()
        pltpu.async_copy(rows_v, out_hbm.at[pl.ds(cbase, CH)], osem).wait()
        return ()

    lax.fori_loop(0, NCHUNK, chunk_body, ())


@jax.jit
def _impl(embed_frozen, weights_train, idx):
    table = jnp.concatenate((embed_frozen, weights_train), axis=0)
    idx2 = idx.reshape(B).astype(jnp.int32)
    out = _gather_kernel(table, idx2)
    return out.reshape(BATCH, HIST, D)


def kernel(embed_frozen, weights_train, idx):
    return _impl(embed_frozen, weights_train, idx)


# speedup vs baseline: 1.0869x; 1.0869x over previous
import functools
import jax
import jax.numpy as jnp
from jax import lax
from jax.experimental import pallas as pl
from jax.experimental.pallas import tpu as pltpu
from jax.experimental.pallas import tpu_sc as plsc

VOCAB = 100000
NADD = 1024
D = 64
BATCH = 4096
HIST = 200
B = BATCH * HIST
NC, NS = 2, 16
NW = NC * NS
BPW = B // NW
CH = 512
NCHUNK = BPW // CH
GW = 128
NSUB = CH // GW

_mesh = plsc.VectorSubcoreMesh(core_axis_name="c", subcore_axis_name="s")


@functools.partial(
    pl.kernel,
    mesh=_mesh,
    out_type=jax.ShapeDtypeStruct((B, D), jnp.float32),
    scratch_types=[
        pltpu.VMEM((CH,), jnp.int32),
        pltpu.VMEM((CH, D), jnp.float32),
        pltpu.SemaphoreType.DMA,
        pltpu.SemaphoreType.DMA,
    ],
    compiler_params=pltpu.CompilerParams(use_tc_tiling_on_sc=False),
)
def _gather_kernel(table_hbm, idx_hbm, out_hbm, idx_v, rows_v, gsem, osem):
    wid = lax.axis_index("s") * NC + lax.axis_index("c")
    base = wid * BPW

    def chunk_body(c, _):
        cbase = base + c * CH
        pltpu.sync_copy(idx_hbm.at[pl.ds(cbase, CH)], idx_v)
        for j in range(NSUB):
            pltpu.async_copy(
                table_hbm.at[idx_v.at[pl.ds(j * GW, GW)]],
                rows_v.at[pl.ds(j * GW, GW)],
                gsem,
            )
        for j in range(NSUB):
            pltpu.make_async_copy(
                table_hbm.at[idx_v.at[pl.ds(j * GW, GW)]],
                rows_v.at[pl.ds(j * GW, GW)],
                gsem,
            ).wait()
        pltpu.async_copy(rows_v, out_hbm.at[pl.ds(cbase, CH)], osem).wait()
        return ()

    lax.fori_loop(0, NCHUNK, chunk_body, ())


@jax.jit
def _impl(embed_frozen, weights_train, idx):
    table = jnp.concatenate((embed_frozen, weights_train), axis=0)
    idx2 = idx.reshape(B).astype(jnp.int32)
    out = _gather_kernel(table, idx2)
    return out.reshape(BATCH, HIST, D)


def kernel(embed_frozen, weights_train, idx):
    return _impl(embed_frozen, weights_train, idx)


# speedup vs baseline: 1.4059x; 1.2935x over previous
import functools
import jax
import jax.numpy as jnp
from jax import lax
from jax.experimental import pallas as pl
from jax.experimental.pallas import tpu as pltpu
from jax.experimental.pallas import tpu_sc as plsc

VOCAB = 100000
NADD = 1024
D = 64
BATCH = 4096
HIST = 200
B = BATCH * HIST
NC, NS = 2, 16
NW = NC * NS
BPW = B // NW
CH = 512
NCHUNK = BPW // CH
GW = 128
NSUB = CH // GW

_mesh = plsc.VectorSubcoreMesh(core_axis_name="c", subcore_axis_name="s")


@functools.partial(
    pl.kernel,
    mesh=_mesh,
    out_type=jax.ShapeDtypeStruct((B, D), jnp.float32),
    scratch_types=[
        pltpu.VMEM((2, CH), jnp.int32),
        pltpu.VMEM((2, CH, D), jnp.float32),
        pltpu.SemaphoreType.DMA,
        pltpu.SemaphoreType.DMA,
        pltpu.SemaphoreType.DMA,
        pltpu.SemaphoreType.DMA,
        pltpu.SemaphoreType.DMA,
        pltpu.SemaphoreType.DMA,
    ],
    compiler_params=pltpu.CompilerParams(use_tc_tiling_on_sc=False),
)
def _gather_kernel(table_hbm, idx_hbm, out_hbm, idx_v, rows_v,
                   isem0, isem1, gsem0, gsem1, ssem0, ssem1):
    wid = lax.axis_index("s") * NC + lax.axis_index("c")
    base = wid * BPW
    isems = (isem0, isem1)
    gsems = (gsem0, gsem1)
    ssems = (ssem0, ssem1)

    def idx_copy(c, b):
        return pltpu.make_async_copy(
            idx_hbm.at[pl.ds(base + c * CH, CH)], idx_v.at[b], isems[b])

    def gather_copy(j, b):
        return pltpu.make_async_copy(
            table_hbm.at[idx_v.at[b].at[pl.ds(j * GW, GW)]],
            rows_v.at[b].at[pl.ds(j * GW, GW)],
            gsems[b])

    def store_copy(c, b):
        return pltpu.make_async_copy(
            rows_v.at[b], out_hbm.at[pl.ds(base + c * CH, CH)], ssems[b])

    def fire_gathers(b):
        for j in range(NSUB):
            gather_copy(j, b).start()

    def drain_gathers(b):
        for j in range(NSUB):
            gather_copy(j, b).wait()

    idx_copy(0, 0).start()
    idx_copy(0, 0).wait()
    fire_gathers(0)
    idx_copy(1, 1).start()
    idx_copy(1, 1).wait()
    fire_gathers(1)
    drain_gathers(0)
    store_copy(0, 0).start()
    idx_copy(2, 0).start()

    def pair_body(g, _):
        for b in range(2):
            c = 2 * g + b
            idx_copy(c, b).wait()
            store_copy(c - 2, b).wait()
            fire_gathers(b)
            drain_gathers(1 - b)
            store_copy(c - 1, 1 - b).start()
            nxt = c + 1
            nxt = jnp.where(nxt == NCHUNK, 0, nxt)
            idx_copy(nxt, 1 - b).start()
        return ()

    lax.fori_loop(1, NCHUNK // 2, pair_body, ())

    drain_gathers(1)
    store_copy(NCHUNK - 1, 1).start()
    store_copy(NCHUNK - 2, 0).wait()
    store_copy(NCHUNK - 1, 1).wait()
    idx_copy(0, 0).wait()


@jax.jit
def _impl(embed_frozen, weights_train, idx):
    table = jnp.concatenate((embed_frozen, weights_train), axis=0)
    idx2 = idx.reshape(B).astype(jnp.int32)
    out = _gather_kernel(table, idx2)
    return out.reshape(BATCH, HIST, D)


def kernel(embed_frozen, weights_train, idx):
    return _impl(embed_frozen, weights_train, idx)
